# baseline (speedup 1.0000x reference)
import functools

import jax
import jax.numpy as jnp
from jax import lax
from jax.experimental import pallas as pl
from jax.experimental.pallas import tpu as pltpu
from jax.experimental.pallas import tpu_sc as plsc

_B = 16384
_HIDDEN = 128
_MAX_SLIP = 0.4
_MAX_GUESS = 0.4
_T = 50.0

_NC = 2
_NS = 16
_NW = _NC * _NS
_B_PER_W = _B // _NW
_UNROLL = 8


def _sc_gather(user, item, theta_table, slip_table, guess_table):
    mesh = plsc.VectorSubcoreMesh(core_axis_name="c", subcore_axis_name="s")

    @functools.partial(
        pl.kernel,
        out_type=(
            jax.ShapeDtypeStruct((_B, _HIDDEN), jnp.float32),
            jax.ShapeDtypeStruct((_B, 1), jnp.float32),
            jax.ShapeDtypeStruct((_B, 1), jnp.float32),
        ),
        mesh=mesh,
        scratch_types=[
            pltpu.VMEM((_B_PER_W,), jnp.int32),
            pltpu.VMEM((_B_PER_W, _HIDDEN), jnp.float32),
            pltpu.VMEM((_B_PER_W,), jnp.int32),
            pltpu.SemaphoreType.DMA,
            pltpu.SemaphoreType.DMA,
        ],
    )
    def gather_kernel(user_hbm, item_hbm, theta_hbm, slip_hbm, guess_hbm,
                      theta_out, slip_out, guess_out,
                      uidx_v, rows_v, iidx_v, sem_t, sem_s):
        wid = lax.axis_index("s") * _NC + lax.axis_index("c")
        base = wid * _B_PER_W
        pltpu.sync_copy(user_hbm.at[pl.ds(base, _B_PER_W)], uidx_v)
        pltpu.sync_copy(item_hbm.at[pl.ds(base, _B_PER_W)], iidx_v)
        cp_t = pltpu.async_copy(theta_hbm.at[uidx_v], rows_v, sem_t)

        @pl.loop(0, _B_PER_W, step=16)
        def _(j):
            v = iidx_v[pl.ds(j, 16)]
            for k in range(16):
                idx = v[k]
                pltpu.async_copy(
                    slip_hbm.at[pl.ds(idx, 1), :],
                    slip_out.at[pl.ds(base + j + k, 1), :], sem_s)
                pltpu.async_copy(
                    guess_hbm.at[pl.ds(idx, 1), :],
                    guess_out.at[pl.ds(base + j + k, 1), :], sem_s)

        @pl.loop(0, _B_PER_W, step=16)
        def _(j):
            v = iidx_v[pl.ds(j, 16)]
            for k in range(16):
                idx = v[k]
                pltpu.make_async_copy(
                    slip_hbm.at[pl.ds(idx, 1), :],
                    slip_out.at[pl.ds(base + j + k, 1), :], sem_s).wait()
                pltpu.make_async_copy(
                    guess_hbm.at[pl.ds(idx, 1), :],
                    guess_out.at[pl.ds(base + j + k, 1), :], sem_s).wait()

        cp_t.wait()
        pltpu.sync_copy(rows_v, theta_out.at[pl.ds(base, _B_PER_W)])

    return gather_kernel(user, item, theta_table, slip_table, guess_table)


def _score_block(theta_ref, kn_ref, s_ref, g_ref, out_ref):
    th = theta_ref[...]
    kn = kn_ref[...]
    n = jnp.sum(kn * (jax.nn.sigmoid(th) - 0.5), axis=1, keepdims=True)
    s = jax.nn.sigmoid(n * (1.0 / _T))
    slip = jax.nn.sigmoid(s_ref[...]) * _MAX_SLIP
    guess = jax.nn.sigmoid(g_ref[...]) * _MAX_GUESS
    out_ref[...] = (1.0 - slip) * s + guess * (1.0 - s)


def kernel(user, item, knowledge, theta_table, slip_table, guess_table):
    theta_g, slip_g, guess_g = _sc_gather(
        user, item, theta_table, slip_table, guess_table)
    rows = 2048
    out = pl.pallas_call(
        _score_block,
        grid=(_B // rows,),
        in_specs=[
            pl.BlockSpec((rows, _HIDDEN), lambda i: (i, 0)),
            pl.BlockSpec((rows, _HIDDEN), lambda i: (i, 0)),
            pl.BlockSpec((rows, 1), lambda i: (i, 0)),
            pl.BlockSpec((rows, 1), lambda i: (i, 0)),
        ],
        out_specs=pl.BlockSpec((rows, 1), lambda i: (i, 0)),
        out_shape=jax.ShapeDtypeStruct((_B, 1), jnp.float32),
    )(theta_g, knowledge, slip_g, guess_g)
    return out.reshape(_B)


# speedup vs baseline: 3.7536x; 3.7536x over previous
import dataclasses
import functools

import jax
import jax.numpy as jnp
from jax import lax
from jax.experimental import pallas as pl
from jax.experimental.pallas import tpu as pltpu
from jax.experimental.pallas import tpu_sc as plsc

_B = 16384
_HIDDEN = 128
_ITEM_NUM = 100000
_MAX_SLIP = 0.4
_MAX_GUESS = 0.4
_T = 50.0

_NC = 2
_NS = 16
_NW = _NC * _NS
_B_PER_W = _B // _NW
_T_PER_W = 3200
_CHUNK = 400
_ROWS_PER_W = _T_PER_W * 2 // 128
_SLOT = 56
_SG_ROWS = _NW * _SLOT


def _sc_build_sg(slip_table, guess_table):
    mesh = plsc.VectorSubcoreMesh(core_axis_name="c", subcore_axis_name="s")

    @functools.partial(
        pl.kernel,
        out_type=jax.ShapeDtypeStruct((_SG_ROWS, 128), jnp.float32),
        mesh=mesh,
        scratch_types=[
            pltpu.VMEM((_CHUNK, 1), jnp.float32),
            pltpu.VMEM((_CHUNK, 1), jnp.float32),
            pltpu.VMEM((_SLOT, 128), jnp.float32),
        ],
        compiler_params=dataclasses.replace(
            pltpu.CompilerParams(), needs_layout_passes=False),
    )
    def build_kernel(slip_hbm, guess_hbm, sg_out, s_v, g_v, b_v):
        wid = lax.axis_index("s") * _NC + lax.axis_index("c")
        tbase = wid * _T_PER_W
        n_chunks = jnp.where(wid == _NW - 1, 2, _T_PER_W // _CHUNK)
        i16 = lax.iota(jnp.int32, 16)
        zeros16 = i16 * 0

        @pl.loop(0, n_chunks)
        def _(c):
            off = pl.multiple_of(tbase + c * _CHUNK, 8)
            pltpu.sync_copy(slip_hbm.at[pl.ds(off, _CHUNK), :], s_v)
            pltpu.sync_copy(guess_hbm.at[pl.ds(off, _CHUNK), :], g_v)

            @pl.loop(0, _CHUNK // 16)
            def _(j):
                p0 = (c * _CHUNK + j * 16) * 2
                r = p0 // 128
                col0 = p0 % 128
                rows = zeros16 + r
                cols = col0 + 2 * i16
                idx = j * 16 + i16
                vs = plsc.load_gather(s_v, [idx, zeros16])
                plsc.store_scatter(b_v, [rows, cols], vs)
                vg = plsc.load_gather(g_v, [idx, zeros16])
                plsc.store_scatter(b_v, [rows, cols + 1], vg)

        obase = pl.multiple_of(wid * _SLOT, 8)

        @pl.when(wid < _NW - 1)
        def _():
            pltpu.sync_copy(b_v, sg_out.at[pl.ds(obase, _SLOT), :])

        @pl.when(wid == _NW - 1)
        def _():
            pltpu.sync_copy(b_v.at[pl.ds(0, 16), :],
                            sg_out.at[pl.ds(obase, 16), :])

    return build_kernel(slip_table, guess_table)


def _sc_gather(user, sg_row_idx, theta_table, sg_flat):
    mesh = plsc.VectorSubcoreMesh(core_axis_name="c", subcore_axis_name="s")

    @functools.partial(
        pl.kernel,
        out_type=(
            jax.ShapeDtypeStruct((_B, _HIDDEN), jnp.float32),
            jax.ShapeDtypeStruct((_B, _HIDDEN), jnp.float32),
        ),
        mesh=mesh,
        scratch_types=[
            pltpu.VMEM((_B_PER_W,), jnp.int32),
            pltpu.VMEM((_B_PER_W, _HIDDEN), jnp.float32),
            pltpu.VMEM((_B_PER_W,), jnp.int32),
            pltpu.VMEM((_B_PER_W // 2, _HIDDEN), jnp.float32),
            pltpu.SemaphoreType.DMA,
            pltpu.SemaphoreType.DMA,
        ],
    )
    def gather_kernel(user_hbm, sgi_hbm, theta_hbm, sg_hbm,
                      theta_out, sg_out,
                      uidx_v, rows_v, iidx_v, sg_v, sem_t, sem_s):
        wid = lax.axis_index("s") * _NC + lax.axis_index("c")
        base = wid * _B_PER_W
        pltpu.sync_copy(user_hbm.at[pl.ds(base, _B_PER_W)], uidx_v)
        pltpu.sync_copy(sgi_hbm.at[pl.ds(base, _B_PER_W)], iidx_v)
        cp_t = pltpu.async_copy(theta_hbm.at[uidx_v], rows_v, sem_t)
        half = _B_PER_W // 2

        @pl.loop(0, 2)
        def _(h):
            off = h * half
            cp_s = pltpu.async_copy(
                sg_hbm.at[iidx_v.at[pl.ds(off, half)]], sg_v, sem_s)
            cp_s.wait()
            pltpu.sync_copy(sg_v, sg_out.at[pl.ds(base + off, half)])

        cp_t.wait()
        pltpu.sync_copy(rows_v, theta_out.at[pl.ds(base, _B_PER_W)])

    return gather_kernel(user, sg_row_idx, theta_table, sg_flat)


def _score_block(theta_ref, kn_ref, sgrow_ref, lane_ref, out_ref):
    th = theta_ref[...]
    kn = kn_ref[...]
    n = jnp.sum(kn * (jax.nn.sigmoid(th) - 0.5), axis=1, keepdims=True)
    s = jax.nn.sigmoid(n * (1.0 / _T))

    sgrow = sgrow_ref[...]
    lane0 = lane_ref[...]
    lanes = lax.broadcasted_iota(jnp.int32, sgrow.shape, 1)
    slip_raw = jnp.sum(jnp.where(lanes == lane0, sgrow, 0.0), axis=1,
                       keepdims=True)
    guess_raw = jnp.sum(jnp.where(lanes == lane0 + 1, sgrow, 0.0), axis=1,
                        keepdims=True)
    slip = jax.nn.sigmoid(slip_raw) * _MAX_SLIP
    guess = jax.nn.sigmoid(guess_raw) * _MAX_GUESS
    out_ref[...] = (1.0 - slip) * s + guess * (1.0 - s)


def kernel(user, item, knowledge, theta_table, slip_table, guess_table):
    sg_flat = _sc_build_sg(slip_table, guess_table)

    w = item // _T_PER_W
    p = (item % _T_PER_W) * 2
    sg_row_idx = (w * _SLOT + p // 128).astype(jnp.int32)
    lane0 = (p % 128).astype(jnp.int32).reshape(_B, 1)

    theta_g, sg_g = _sc_gather(user, sg_row_idx, theta_table, sg_flat)

    rows = 2048
    out = pl.pallas_call(
        _score_block,
        grid=(_B // rows,),
        in_specs=[
            pl.BlockSpec((rows, _HIDDEN), lambda i: (i, 0)),
            pl.BlockSpec((rows, _HIDDEN), lambda i: (i, 0)),
            pl.BlockSpec((rows, _HIDDEN), lambda i: (i, 0)),
            pl.BlockSpec((rows, 1), lambda i: (i, 0)),
        ],
        out_specs=pl.BlockSpec((rows, 1), lambda i: (i, 0)),
        out_shape=jax.ShapeDtypeStruct((_B, 1), jnp.float32),
    )(theta_g, knowledge, sg_g, lane0)
    return out.reshape(_B)


# speedup vs baseline: 8.4140x; 2.2416x over previous
import dataclasses
import functools

import jax
import jax.numpy as jnp
from jax import lax
from jax.experimental import pallas as pl
from jax.experimental.pallas import tpu as pltpu
from jax.experimental.pallas import tpu_sc as plsc

_B = 16384
_HIDDEN = 128
_ITEM_NUM = 100000
_MAX_SLIP = 0.4
_MAX_GUESS = 0.4
_T = 50.0

_NC = 2
_NS = 16
_NW = _NC * _NS
_B_PER_W = _B // _NW
_T_PER_W = 3200
_CHUNK = 400
_ROWS_PER_W = _T_PER_W * 2 // 128
_SLOT = 56
_SG_ROWS = _NW * _SLOT


def _sc_build_sg(slip_flat, guess_flat):
    mesh = plsc.VectorSubcoreMesh(core_axis_name="c", subcore_axis_name="s")

    @functools.partial(
        pl.kernel,
        out_type=jax.ShapeDtypeStruct((_SG_ROWS, 128), jnp.float32),
        mesh=mesh,
        scratch_types=[
            pltpu.VMEM((_T_PER_W,), jnp.float32),
            pltpu.VMEM((_T_PER_W,), jnp.float32),
            pltpu.VMEM((_SLOT, 128), jnp.float32),
        ],
        compiler_params=dataclasses.replace(
            pltpu.CompilerParams(), needs_layout_passes=False),
    )
    def build_kernel(slip_hbm, guess_hbm, sg_out, s_v, g_v, b_v):
        wid = lax.axis_index("s") * _NC + lax.axis_index("c")
        tbase = pl.multiple_of(wid * _T_PER_W, 8)
        pltpu.sync_copy(slip_hbm.at[pl.ds(tbase, _T_PER_W)], s_v)
        pltpu.sync_copy(guess_hbm.at[pl.ds(tbase, _T_PER_W)], g_v)
        i16 = lax.iota(jnp.int32, 16)

        @pl.loop(0, _T_PER_W // 16)
        def _(j):
            p0 = j * 32
            rows = (i16 * 0) + (p0 // 128)
            cols = (p0 % 128) + 2 * i16
            vs = s_v[pl.ds(j * 16, 16)]
            vg = g_v[pl.ds(j * 16, 16)]
            plsc.store_scatter(b_v, [rows, cols], vs)
            plsc.store_scatter(b_v, [rows, cols + 1], vg)

        obase = pl.multiple_of(wid * _SLOT, 8)
        pltpu.sync_copy(b_v, sg_out.at[pl.ds(obase, _SLOT), :])

    return build_kernel(slip_flat, guess_flat)


def _sc_gather(user, sg_row_idx, theta_table, sg_flat):
    mesh = plsc.VectorSubcoreMesh(core_axis_name="c", subcore_axis_name="s")

    @functools.partial(
        pl.kernel,
        out_type=(
            jax.ShapeDtypeStruct((_B, _HIDDEN), jnp.float32),
            jax.ShapeDtypeStruct((_B, _HIDDEN), jnp.float32),
        ),
        mesh=mesh,
        scratch_types=[
            pltpu.VMEM((_B_PER_W,), jnp.int32),
            pltpu.VMEM((_B_PER_W, _HIDDEN), jnp.float32),
            pltpu.VMEM((_B_PER_W,), jnp.int32),
            pltpu.VMEM((_B_PER_W // 2, _HIDDEN), jnp.float32),
            pltpu.SemaphoreType.DMA,
            pltpu.SemaphoreType.DMA,
        ],
    )
    def gather_kernel(user_hbm, sgi_hbm, theta_hbm, sg_hbm,
                      theta_out, sg_out,
                      uidx_v, rows_v, iidx_v, sg_v, sem_t, sem_s):
        wid = lax.axis_index("s") * _NC + lax.axis_index("c")
        base = wid * _B_PER_W
        pltpu.sync_copy(user_hbm.at[pl.ds(base, _B_PER_W)], uidx_v)
        pltpu.sync_copy(sgi_hbm.at[pl.ds(base, _B_PER_W)], iidx_v)
        cp_t = pltpu.async_copy(theta_hbm.at[uidx_v], rows_v, sem_t)
        half = _B_PER_W // 2

        @pl.loop(0, 2)
        def _(h):
            off = h * half
            cp_s = pltpu.async_copy(
                sg_hbm.at[iidx_v.at[pl.ds(off, half)]], sg_v, sem_s)
            cp_s.wait()
            pltpu.sync_copy(sg_v, sg_out.at[pl.ds(base + off, half)])

        cp_t.wait()
        pltpu.sync_copy(rows_v, theta_out.at[pl.ds(base, _B_PER_W)])

    return gather_kernel(user, sg_row_idx, theta_table, sg_flat)


def _score_block(theta_ref, kn_ref, sgrow_ref, lane_ref, out_ref):
    th = theta_ref[...]
    kn = kn_ref[...]
    n = jnp.sum(kn * (jax.nn.sigmoid(th) - 0.5), axis=1, keepdims=True)
    s = jax.nn.sigmoid(n * (1.0 / _T))

    sgrow = sgrow_ref[...]
    lane0 = lane_ref[...]
    lanes = lax.broadcasted_iota(jnp.int32, sgrow.shape, 1)
    slip_raw = jnp.sum(jnp.where(lanes == lane0, sgrow, 0.0), axis=1,
                       keepdims=True)
    guess_raw = jnp.sum(jnp.where(lanes == lane0 + 1, sgrow, 0.0), axis=1,
                        keepdims=True)
    slip = jax.nn.sigmoid(slip_raw) * _MAX_SLIP
    guess = jax.nn.sigmoid(guess_raw) * _MAX_GUESS
    out_ref[...] = (1.0 - slip) * s + guess * (1.0 - s)


def kernel(user, item, knowledge, theta_table, slip_table, guess_table):
    pad = _NW * _T_PER_W - _ITEM_NUM
    slip_flat = jnp.pad(slip_table.reshape(_ITEM_NUM), (0, pad))
    guess_flat = jnp.pad(guess_table.reshape(_ITEM_NUM), (0, pad))
    sg_flat = _sc_build_sg(slip_flat, guess_flat)

    w = item // _T_PER_W
    p = (item % _T_PER_W) * 2
    sg_row_idx = (w * _SLOT + p // 128).astype(jnp.int32)
    lane0 = (p % 128).astype(jnp.int32).reshape(_B, 1)

    theta_g, sg_g = _sc_gather(user, sg_row_idx, theta_table, sg_flat)

    rows = 2048
    out = pl.pallas_call(
        _score_block,
        grid=(_B // rows,),
        in_specs=[
            pl.BlockSpec((rows, _HIDDEN), lambda i: (i, 0)),
            pl.BlockSpec((rows, _HIDDEN), lambda i: (i, 0)),
            pl.BlockSpec((rows, _HIDDEN), lambda i: (i, 0)),
            pl.BlockSpec((rows, 1), lambda i: (i, 0)),
        ],
        out_specs=pl.BlockSpec((rows, 1), lambda i: (i, 0)),
        out_shape=jax.ShapeDtypeStruct((_B, 1), jnp.float32),
    )(theta_g, knowledge, sg_g, lane0)
    return out.reshape(_B)


# speedup vs baseline: 9.8109x; 1.1660x over previous
import dataclasses
import functools

import jax
import jax.numpy as jnp
from jax import lax
from jax.experimental import pallas as pl
from jax.experimental.pallas import tpu as pltpu
from jax.experimental.pallas import tpu_sc as plsc

_B = 16384
_HIDDEN = 128
_ITEM_NUM = 100000
_MAX_SLIP = 0.4
_MAX_GUESS = 0.4
_T = 50.0

_NC = 2
_NS = 16
_NW = _NC * _NS
_B_PER_W = _B // _NW
_T_PER_W = 3200
_CHUNK = 400
_ROWS_PER_W = _T_PER_W * 2 // 128
_SLOT = 56
_SG_ROWS = _NW * _SLOT


def _sc_build_sg(slip_flat, guess_flat):
    mesh = plsc.VectorSubcoreMesh(core_axis_name="c", subcore_axis_name="s")

    @functools.partial(
        pl.kernel,
        out_type=jax.ShapeDtypeStruct((_SG_ROWS, 128), jnp.float32),
        mesh=mesh,
        scratch_types=[
            pltpu.VMEM((_T_PER_W,), jnp.float32),
            pltpu.VMEM((_T_PER_W,), jnp.float32),
            pltpu.VMEM((_SLOT, 128), jnp.float32),
        ],
        compiler_params=dataclasses.replace(
            pltpu.CompilerParams(), needs_layout_passes=False),
    )
    def build_kernel(slip_hbm, guess_hbm, sg_out, s_v, g_v, b_v):
        wid = lax.axis_index("s") * _NC + lax.axis_index("c")
        tbase = pl.multiple_of(wid * _T_PER_W, 8)
        pltpu.sync_copy(slip_hbm.at[pl.ds(tbase, _T_PER_W)], s_v)
        pltpu.sync_copy(guess_hbm.at[pl.ds(tbase, _T_PER_W)], g_v)
        i16 = lax.iota(jnp.int32, 16)

        @pl.loop(0, _T_PER_W // 16)
        def _(j):
            p0 = j * 32
            rows = (i16 * 0) + (p0 // 128)
            cols = (p0 % 128) + 2 * i16
            vs = s_v[pl.ds(j * 16, 16)]
            vg = g_v[pl.ds(j * 16, 16)]
            plsc.store_scatter(b_v, [rows, cols], vs)
            plsc.store_scatter(b_v, [rows, cols + 1], vg)

        obase = pl.multiple_of(wid * _SLOT, 8)
        pltpu.sync_copy(b_v, sg_out.at[pl.ds(obase, _SLOT), :])

    return build_kernel(slip_flat, guess_flat)


def _sc_gather(user, sg_row_idx, lane0, theta_table, sg_flat):
    mesh = plsc.VectorSubcoreMesh(core_axis_name="c", subcore_axis_name="s")

    @functools.partial(
        pl.kernel,
        out_type=(
            jax.ShapeDtypeStruct((_B, _HIDDEN), jnp.float32),
            jax.ShapeDtypeStruct((_B,), jnp.float32),
            jax.ShapeDtypeStruct((_B,), jnp.float32),
        ),
        mesh=mesh,
        scratch_types=[
            pltpu.VMEM((_B_PER_W,), jnp.int32),
            pltpu.VMEM((_B_PER_W, _HIDDEN), jnp.float32),
            pltpu.VMEM((_B_PER_W,), jnp.int32),
            pltpu.VMEM((_B_PER_W,), jnp.int32),
            pltpu.VMEM((_B_PER_W // 2, _HIDDEN), jnp.float32),
            pltpu.VMEM((_B_PER_W,), jnp.float32),
            pltpu.VMEM((_B_PER_W,), jnp.float32),
            pltpu.SemaphoreType.DMA,
            pltpu.SemaphoreType.DMA,
        ],
        compiler_params=dataclasses.replace(
            pltpu.CompilerParams(), needs_layout_passes=False),
    )
    def gather_kernel(user_hbm, sgi_hbm, lane_hbm, theta_hbm, sg_hbm,
                      theta_out, slip_out, guess_out,
                      uidx_v, rows_v, iidx_v, lane_v, sg_v, s_v, g_v,
                      sem_t, sem_s):
        wid = lax.axis_index("s") * _NC + lax.axis_index("c")
        base = wid * _B_PER_W
        pltpu.sync_copy(user_hbm.at[pl.ds(base, _B_PER_W)], uidx_v)
        pltpu.sync_copy(sgi_hbm.at[pl.ds(base, _B_PER_W)], iidx_v)
        pltpu.sync_copy(lane_hbm.at[pl.ds(base, _B_PER_W)], lane_v)
        cp_t = pltpu.async_copy(theta_hbm.at[uidx_v], rows_v, sem_t)
        half = _B_PER_W // 2
        i16 = lax.iota(jnp.int32, 16)

        @pl.loop(0, 2)
        def _(h):
            off = h * half
            cp_s = pltpu.async_copy(
                sg_hbm.at[iidx_v.at[pl.ds(off, half)]], sg_v, sem_s)
            cp_s.wait()

            @pl.loop(0, half // 16)
            def _(k):
                rows = k * 16 + i16
                cols = lane_v[pl.ds(off + k * 16, 16)]
                s_v[pl.ds(off + k * 16, 16)] = plsc.load_gather(
                    sg_v, [rows, cols])
                g_v[pl.ds(off + k * 16, 16)] = plsc.load_gather(
                    sg_v, [rows, cols + 1])

        pltpu.sync_copy(s_v, slip_out.at[pl.ds(base, _B_PER_W)])
        pltpu.sync_copy(g_v, guess_out.at[pl.ds(base, _B_PER_W)])
        cp_t.wait()
        pltpu.sync_copy(rows_v, theta_out.at[pl.ds(base, _B_PER_W)])

    return gather_kernel(user, sg_row_idx, lane0, theta_table, sg_flat)


def _score_block(theta_ref, kn_ref, s_ref, g_ref, out_ref):
    th = theta_ref[...]
    kn = kn_ref[...]
    n = jnp.sum(kn * (jax.nn.sigmoid(th) - 0.5), axis=1)
    s = jax.nn.sigmoid(n * (1.0 / _T))
    slip = jax.nn.sigmoid(s_ref[...]) * _MAX_SLIP
    guess = jax.nn.sigmoid(g_ref[...]) * _MAX_GUESS
    out_ref[...] = (1.0 - slip) * s + guess * (1.0 - s)


def kernel(user, item, knowledge, theta_table, slip_table, guess_table):
    pad = _NW * _T_PER_W - _ITEM_NUM
    slip_flat = jnp.pad(slip_table.reshape(_ITEM_NUM), (0, pad))
    guess_flat = jnp.pad(guess_table.reshape(_ITEM_NUM), (0, pad))
    sg_flat = _sc_build_sg(slip_flat, guess_flat)

    w = item // _T_PER_W
    p = (item % _T_PER_W) * 2
    sg_row_idx = (w * _SLOT + p // 128).astype(jnp.int32)
    lane0 = (p % 128).astype(jnp.int32)

    theta_g, slip_g, guess_g = _sc_gather(
        user, sg_row_idx, lane0, theta_table, sg_flat)

    rows = 2048
    out = pl.pallas_call(
        _score_block,
        grid=(_B // rows,),
        in_specs=[
            pl.BlockSpec((rows, _HIDDEN), lambda i: (i, 0)),
            pl.BlockSpec((rows, _HIDDEN), lambda i: (i, 0)),
            pl.BlockSpec((rows,), lambda i: (i,)),
            pl.BlockSpec((rows,), lambda i: (i,)),
        ],
        out_specs=pl.BlockSpec((rows,), lambda i: (i,)),
        out_shape=jax.ShapeDtypeStruct((_B,), jnp.float32),
    )(theta_g, knowledge, slip_g, guess_g)
    return out
